# Initial kernel scaffold; baseline (speedup 1.0000x reference)
import jax
import jax.numpy as jnp
from jax.experimental import pallas as pl


def kernel(state, neighb_dirs):
    raise NotImplementedError("write your pallas kernel here")


# speedup vs baseline: 88.2014x; 88.2014x over previous
import jax
import jax.numpy as jnp
import numpy as np
from jax.experimental import pallas as pl

_N_NEIGH = 27
_SIGNAL_DIM = 16
_N_DIRS = 8
_DIRS_DIM = 3
_SH_DIM = _SIGNAL_DIM + 1
_SH_END = _SH_DIM * _N_NEIGH
_STATE_DIM = _SH_END + _N_DIRS * _DIRS_DIM
_OUT_DIM = 41

_PERM = np.concatenate([np.arange(0, 4), [16], np.arange(4, 16),
                        np.arange(17, 41)]).astype(np.int32)

_BLOCK_ROWS = 2000


def _matmul_body(x_ref, m_ref, o_ref):
    o_ref[...] = jnp.dot(x_ref[...], m_ref[...],
                         preferred_element_type=jnp.float32)


def _build_m(neighb_dirs):
    rel = neighb_dirs[1:] - neighb_dirs[0:1]
    w = jnp.exp(-jnp.sqrt(jnp.sum(rel * rel, axis=-1)))
    a = jnp.concatenate([jnp.sum(w)[None], w]) / _N_NEIGH

    m = jnp.zeros((_STATE_DIM, _OUT_DIM), dtype=jnp.float32)
    sh_js = np.nonzero(_PERM < _SH_DIM)[0]
    rows = (np.arange(_N_NEIGH)[:, None] * _SH_DIM +
            _PERM[sh_js][None, :]).reshape(-1)
    cols = np.broadcast_to(sh_js[None, :], (_N_NEIGH, len(sh_js))).reshape(-1)
    vals = jnp.broadcast_to(a[:, None], (_N_NEIGH, len(sh_js))).reshape(-1)
    m = m.at[rows, cols].set(vals)
    dir_js = np.nonzero(_PERM >= _SH_DIM)[0]
    m = m.at[_SH_END + (_PERM[dir_js] - _SH_DIM), dir_js].set(2.0 * a[0])
    return m


def kernel(state, neighb_dirs):
    b = state.shape[0]
    m = _build_m(neighb_dirs)
    grid = (b // _BLOCK_ROWS,)
    return pl.pallas_call(
        _matmul_body,
        grid=grid,
        in_specs=[
            pl.BlockSpec((_BLOCK_ROWS, _STATE_DIM), lambda i: (i, 0)),
            pl.BlockSpec((_STATE_DIM, _OUT_DIM), lambda i: (0, 0)),
        ],
        out_specs=pl.BlockSpec((_BLOCK_ROWS, _OUT_DIM), lambda i: (i, 0)),
        out_shape=jax.ShapeDtypeStruct((b, _OUT_DIM), jnp.float32),
    )(state, m)


# speedup vs baseline: 89.6642x; 1.0166x over previous
import jax
import jax.numpy as jnp
import numpy as np
from jax.experimental import pallas as pl

_N_NEIGH = 27
_SIGNAL_DIM = 16
_N_DIRS = 8
_DIRS_DIM = 3
_SH_DIM = _SIGNAL_DIM + 1
_SH_END = _SH_DIM * _N_NEIGH
_STATE_DIM = _SH_END + _N_DIRS * _DIRS_DIM
_OUT_DIM = 41

_PERM = np.concatenate([np.arange(0, 4), [16], np.arange(4, 16),
                        np.arange(17, 41)]).astype(np.int32)

_BLOCK_ROWS = 4000


def _matmul_body(x_ref, m_ref, o_ref):
    o_ref[...] = jnp.dot(x_ref[...], m_ref[...],
                         preferred_element_type=jnp.float32)


def _build_m(neighb_dirs):
    rel = neighb_dirs[1:] - neighb_dirs[0:1]
    w = jnp.exp(-jnp.sqrt(jnp.sum(rel * rel, axis=-1)))
    a = jnp.concatenate([jnp.sum(w)[None], w]) / _N_NEIGH

    m = jnp.zeros((_STATE_DIM, _OUT_DIM), dtype=jnp.float32)
    sh_js = np.nonzero(_PERM < _SH_DIM)[0]
    rows = (np.arange(_N_NEIGH)[:, None] * _SH_DIM +
            _PERM[sh_js][None, :]).reshape(-1)
    cols = np.broadcast_to(sh_js[None, :], (_N_NEIGH, len(sh_js))).reshape(-1)
    vals = jnp.broadcast_to(a[:, None], (_N_NEIGH, len(sh_js))).reshape(-1)
    m = m.at[rows, cols].set(vals)
    dir_js = np.nonzero(_PERM >= _SH_DIM)[0]
    m = m.at[_SH_END + (_PERM[dir_js] - _SH_DIM), dir_js].set(2.0 * a[0])
    return m


def kernel(state, neighb_dirs):
    b = state.shape[0]
    m = _build_m(neighb_dirs)
    grid = (b // _BLOCK_ROWS,)
    return pl.pallas_call(
        _matmul_body,
        grid=grid,
        in_specs=[
            pl.BlockSpec((_BLOCK_ROWS, _STATE_DIM), lambda i: (i, 0)),
            pl.BlockSpec((_STATE_DIM, _OUT_DIM), lambda i: (0, 0)),
        ],
        out_specs=pl.BlockSpec((_BLOCK_ROWS, _OUT_DIM), lambda i: (i, 0)),
        out_shape=jax.ShapeDtypeStruct((b, _OUT_DIM), jnp.float32),
    )(state, m)


# speedup vs baseline: 89.7551x; 1.0010x over previous
import jax
import jax.numpy as jnp
import numpy as np
from jax.experimental import pallas as pl
from jax.experimental.pallas import tpu as pltpu

_N_NEIGH = 27
_SIGNAL_DIM = 16
_N_DIRS = 8
_DIRS_DIM = 3
_SH_DIM = _SIGNAL_DIM + 1
_SH_END = _SH_DIM * _N_NEIGH
_STATE_DIM = _SH_END + _N_DIRS * _DIRS_DIM
_OUT_DIM = 41

_PERM = np.concatenate([np.arange(0, 4), [16], np.arange(4, 16),
                        np.arange(17, 41)]).astype(np.int32)

_BLOCK_ROWS = 4000


def _matmul_body(x_ref, m_ref, o_ref):
    o_ref[...] = jnp.dot(x_ref[...], m_ref[...],
                         preferred_element_type=jnp.float32)


def _build_m(neighb_dirs):
    rel = neighb_dirs[1:] - neighb_dirs[0:1]
    w = jnp.exp(-jnp.sqrt(jnp.sum(rel * rel, axis=-1)))
    a = jnp.concatenate([jnp.sum(w)[None], w]) / _N_NEIGH

    m = jnp.zeros((_STATE_DIM, _OUT_DIM), dtype=jnp.float32)
    sh_js = np.nonzero(_PERM < _SH_DIM)[0]
    rows = (np.arange(_N_NEIGH)[:, None] * _SH_DIM +
            _PERM[sh_js][None, :]).reshape(-1)
    cols = np.broadcast_to(sh_js[None, :], (_N_NEIGH, len(sh_js))).reshape(-1)
    vals = jnp.broadcast_to(a[:, None], (_N_NEIGH, len(sh_js))).reshape(-1)
    m = m.at[rows, cols].set(vals)
    dir_js = np.nonzero(_PERM >= _SH_DIM)[0]
    m = m.at[_SH_END + (_PERM[dir_js] - _SH_DIM), dir_js].set(2.0 * a[0])
    return m


def kernel(state, neighb_dirs):
    b = state.shape[0]
    m = _build_m(neighb_dirs)
    grid = (b // _BLOCK_ROWS,)
    return pl.pallas_call(
        _matmul_body,
        grid=grid,
        in_specs=[
            pl.BlockSpec((_BLOCK_ROWS, _STATE_DIM), lambda i: (i, 0)),
            pl.BlockSpec((_STATE_DIM, _OUT_DIM), lambda i: (0, 0)),
        ],
        out_specs=pl.BlockSpec((_BLOCK_ROWS, _OUT_DIM), lambda i: (i, 0)),
        out_shape=jax.ShapeDtypeStruct((b, _OUT_DIM), jnp.float32),
        compiler_params=pltpu.CompilerParams(
            dimension_semantics=("parallel",)),
    )(state, m)
